# Initial kernel scaffold; baseline (speedup 1.0000x reference)
import jax
import jax.numpy as jnp
from jax.experimental import pallas as pl


def kernel(x_nchw, conv1_w, conv1_shift, conv2_w, conv2_shift, conv3_w, conv3_shift, conv4_w, conv4_shift, conv5_w, conv5_shift, conv6_w, conv6_shift, fc1_w, fc1_b, fc2_w, fc2_b):
    raise NotImplementedError("write your pallas kernel here")


# speedup vs baseline: 1.2094x; 1.2094x over previous
import jax
import jax.numpy as jnp
from jax.experimental import pallas as pl
from jax.experimental.pallas import tpu as pltpu


def _conv1_kernel(p_ref, w_ref, s_ref, o_ref):
    B, H, W, T = p_ref.shape
    C = 32
    P = p_ref[...].reshape(B * H * W, T)
    Y = jnp.dot(P, w_ref[...], preferred_element_type=jnp.float32)
    y = jnp.maximum(jnp.maximum(Y[:, 0 * C:1 * C], Y[:, 1 * C:2 * C]),
                    jnp.maximum(Y[:, 2 * C:3 * C], Y[:, 3 * C:4 * C]))
    y = jnp.maximum(y + s_ref[...], 0.0).astype(o_ref.dtype)
    o_ref[...] = y.reshape(B, H, W, C)


def _conv1(x_nchw, w, shift, B=2):
    n = x_nchw.shape[0]
    x = x_nchw.reshape(n, 192, 192).astype(jnp.bfloat16)
    taps = [x[:, r:r + 188:2, c:c + 188:2]
            for r in range(6) for c in range(6)]
    P = jnp.stack(taps, axis=-1)
    P = jnp.pad(P, ((0, 0), (0, 0), (0, 2), (0, 0)))
    w2 = w.reshape(5, 5, 32)
    wall = jnp.concatenate(
        [jnp.pad(w2, ((a, 1 - a), (b, 1 - b), (0, 0))).reshape(36, 32)
         for a in (0, 1) for b in (0, 1)], axis=1)
    return pl.pallas_call(
        _conv1_kernel,
        out_shape=jax.ShapeDtypeStruct((n, 94, 96, 32), jnp.bfloat16),
        grid=(n // B,),
        in_specs=[
            pl.BlockSpec((B, 94, 96, 36), lambda g: (g, 0, 0, 0)),
            pl.BlockSpec((36, 128), lambda g: (0, 0)),
            pl.BlockSpec((1, 32), lambda g: (0, 0)),
        ],
        out_specs=pl.BlockSpec((B, 94, 96, 32), lambda g: (g, 0, 0, 0)),
        compiler_params=pltpu.CompilerParams(
            dimension_semantics=("parallel",)),
    )(P, wall, shift)


def _make_conv_kernel(k, oh, ow, wpad):
    ph, pw = oh // 2, ow // 2

    def body(x_ref, w_ref, s_ref, o_ref, *pr_refs):
        B, H, W, cin = x_ref.shape
        cout = w_ref.shape[-1]
        X = x_ref[...]
        acc = jnp.zeros((B, oh, ow, cout), jnp.float32)
        for i in range(k):
            Mi = X[:, i:i + oh].reshape(B * oh * W, cin)
            for j in range(k):
                Z = jnp.dot(Mi, w_ref[i * k + j],
                            preferred_element_type=jnp.float32)
                acc = acc + Z.reshape(B, oh, W, cout)[:, :, j:j + ow, :]
        a2 = acc.reshape(B, ph, 2, ow, cout)
        pr = jnp.maximum(a2[:, :, 0], a2[:, :, 1])
        chunks = []
        for c, pr_ref in enumerate(pr_refs):
            lo = c * 128
            hi = min(cout, lo + 128)
            pr_ref[:, :, :, 0:hi - lo] = pr[..., lo:hi]
            pc = jnp.maximum(pr_ref[:, :, pl.ds(0, pw, 2), :],
                             pr_ref[:, :, pl.ds(1, pw, 2), :])
            chunks.append(pc[..., 0:hi - lo])
        pc = chunks[0] if len(chunks) == 1 else jnp.concatenate(chunks, -1)
        y = jnp.maximum(pc + s_ref[...], 0.0).astype(o_ref.dtype)
        o_ref[:, :, 0:pw, :] = y
        if wpad > pw:
            o_ref[:, :, pw:wpad, :] = jnp.zeros((B, ph, wpad - pw, cout),
                                                o_ref.dtype)

    return body


def _conv_stage(x, w, shift, k, oh, ow, wpad, B):
    n, h, W, cin = x.shape
    cout = w.shape[-1]
    ph = oh // 2
    return pl.pallas_call(
        _make_conv_kernel(k, oh, ow, wpad),
        scratch_shapes=[pltpu.VMEM((B, ph, ow, 128), jnp.float32)
                        for _ in range(max(1, cout // 128))],
        out_shape=jax.ShapeDtypeStruct((n, ph, wpad, cout), jnp.bfloat16),
        grid=(n // B,),
        in_specs=[
            pl.BlockSpec((B, h, W, cin), lambda g: (g, 0, 0, 0)),
            pl.BlockSpec((k * k, cin, cout), lambda g: (0, 0, 0)),
            pl.BlockSpec((1, cout), lambda g: (0, 0)),
        ],
        out_specs=pl.BlockSpec((B, ph, wpad, cout), lambda g: (g, 0, 0, 0)),
        compiler_params=pltpu.CompilerParams(
            dimension_semantics=("parallel",)),
    )(x, w, shift)


def _make_conv6_fc_kernel():
    def body(x_ref, w_ref, s_ref, w1_ref, b1_ref, w2_ref, b2_ref, o_ref):
        B, H, W, cin = x_ref.shape
        cout = w_ref.shape[-1]
        X = x_ref[...]
        acc = jnp.zeros((B, 2, 2, cout), jnp.float32)
        for i in range(3):
            Mi = X[:, i:i + 2].reshape(B * 2 * W, cin)
            for j in range(3):
                Z = jnp.dot(Mi, w_ref[i * 3 + j],
                            preferred_element_type=jnp.float32)
                acc = acc + Z.reshape(B, 2, W, cout)[:, :, j:j + 2, :]
        feats = jnp.maximum(jnp.maximum(acc[:, 0, 0], acc[:, 0, 1]),
                            jnp.maximum(acc[:, 1, 0], acc[:, 1, 1]))
        feats = jnp.maximum(feats + s_ref[...], 0.0).astype(jnp.bfloat16)
        h = jnp.dot(feats, w1_ref[...],
                    preferred_element_type=jnp.float32) + b1_ref[...]
        h = jnp.where(h > 0, h, 0.01 * h).astype(jnp.bfloat16)
        o_ref[...] = jnp.dot(h, w2_ref[...],
                             preferred_element_type=jnp.float32) + b2_ref[...]

    return body


def _conv6_fc(x, w, shift, w1, b1, w2, b2, B=32):
    n = x.shape[0]
    return pl.pallas_call(
        _make_conv6_fc_kernel(),
        out_shape=jax.ShapeDtypeStruct((n, 136), jnp.float32),
        grid=(n // B,),
        in_specs=[
            pl.BlockSpec((B, 4, 16, 256), lambda g: (g, 0, 0, 0)),
            pl.BlockSpec((9, 256, 512), lambda g: (0, 0, 0)),
            pl.BlockSpec((1, 512), lambda g: (0, 0)),
            pl.BlockSpec((512, 256), lambda g: (0, 0)),
            pl.BlockSpec((1, 256), lambda g: (0, 0)),
            pl.BlockSpec((256, 136), lambda g: (0, 0)),
            pl.BlockSpec((1, 136), lambda g: (0, 0)),
        ],
        out_specs=pl.BlockSpec((B, 136), lambda g: (g, 0)),
        compiler_params=pltpu.CompilerParams(
            dimension_semantics=("parallel",)),
    )(x, w, shift, w1, b1, w2, b2)


def kernel(x_nchw, conv1_w, conv1_shift, conv2_w, conv2_shift,
           conv3_w, conv3_shift, conv4_w, conv4_shift,
           conv5_w, conv5_shift, conv6_w, conv6_shift,
           fc1_w, fc1_b, fc2_w, fc2_b):
    x = _conv1(x_nchw, conv1_w, conv1_shift)
    x = _conv_stage(x, conv2_w, conv2_shift, 3, 92, 92, 48, B=2)
    x = _conv_stage(x, conv3_w, conv3_shift, 3, 44, 44, 24, B=4)
    x = _conv_stage(x, conv4_w, conv4_shift, 3, 20, 20, 16, B=8)
    x = _conv_stage(x, conv5_w, conv5_shift, 3, 8, 8, 16, B=16)
    return _conv6_fc(x, conv6_w, conv6_shift, fc1_w, fc1_b, fc2_w, fc2_b)
